# baseline (speedup 1.0000x reference)
import jax
import jax.numpy as jnp
from jax.experimental import pallas as pl

H = 8
INF = 16
BOND = 64
ATOM = 128


def _leaky_relu(x):
    return jnp.where(x >= 0, x, 0.01 * x)


def _egat(nfeats, efeats, src, dst, n_nodes, Wnode, bnode, Wni, Wfij, Wnj, attn, bias, oe, on):
    f_ni = nfeats @ Wni.T
    f_nj = nfeats @ Wnj.T
    f_fij = efeats @ Wfij.T
    f_out = f_ni[src] + f_nj[dst] + f_fij + bias
    f_out = _leaky_relu(f_out).reshape(-1, H, oe)
    e = jnp.sum(f_out * attn, axis=-1, keepdims=True)
    emax = jax.ops.segment_max(e, dst, num_segments=n_nodes)
    emax = jnp.where(jnp.isfinite(emax), emax, 0.0)
    ee = jnp.exp(e - emax[dst])
    den = jax.ops.segment_sum(ee, dst, num_segments=n_nodes)
    a = ee / (den[dst] + 1e-16)
    h = (nfeats @ Wnode.T + bnode).reshape(-1, H, on)
    h_out = jax.ops.segment_sum(h[src] * a, dst, num_segments=n_nodes)
    return h_out, f_out


def kernel(node_feats, edge_feats, node_path, edge_path, edge_index_lg, edge_index_gg,
           Wnode1, bnode1, Wni1, Wfij1, Wnj1, attn1, b1,
           Wnode2a, bnode2a, Wni2a, Wfij2a, Wnj2a, attn2a, b2a,
           Wnode2b, bnode2b, Wni2b, Wfij2b, Wnj2b, attn2b, b2b):
    src_g, dst_g = edge_index_gg[0], edge_index_gg[1]
    np_h, ep_f = _egat(node_path, edge_path, src_g, dst_g, node_path.shape[0],
                       Wnode1, bnode1, Wni1, Wfij1, Wnj1, attn1, b1, INF, BOND)
    y2 = jnp.sum(np_h, axis=1)
    ef = jnp.repeat(y2, 2, axis=0)
    nf = node_feats
    src_l, dst_l = edge_index_lg[0], edge_index_lg[1]
    for (Wn, bn, Wi, Wf, Wj, at, bb) in [
        (Wnode2a, bnode2a, Wni2a, Wfij2a, Wnj2a, attn2a, b2a),
        (Wnode2b, bnode2b, Wni2b, Wfij2b, Wnj2b, attn2b, b2b),
    ]:
        nf_h, ef_f = _egat(nf, ef, src_l, dst_l, nf.shape[0], Wn, bn, Wi, Wf, Wj, at, bb, BOND, ATOM)
        nf = jnp.sum(nf_h, axis=1)
        ef = jnp.sum(ef_f, axis=1)
    return nf, ef


# speedup vs baseline: 8.0884x; 8.0884x over previous
import functools

import jax
import jax.numpy as jnp
from jax import lax
from jax.experimental import pallas as pl
from jax.experimental.pallas import tpu as pltpu
from jax.experimental.pallas import tpu_sc as plsc

H = 8
INF = 16
BOND = 64
ATOM = 128
NC = 2
NS = 16
NW = NC * NS
EB = 32
ZR = 25


def _mm_body(x_ref, w_ref, b_ref, o_ref):
    o_ref[...] = lax.dot_general(
        x_ref[...], w_ref[...], (((1,), (1,)), ((), ())),
        preferred_element_type=jnp.float32) + b_ref[...]


def _matmul(x, w, b=None, bm=1000):
    M, K = x.shape
    No = w.shape[0]
    b2 = jnp.zeros((1, No), jnp.float32) if b is None else jnp.reshape(b, (1, No))
    return pl.pallas_call(
        _mm_body,
        grid=(M // bm,),
        in_specs=[pl.BlockSpec((bm, K), lambda i: (i, 0)),
                  pl.BlockSpec((No, K), lambda i: (0, 0)),
                  pl.BlockSpec((1, No), lambda i: (0, 0))],
        out_specs=pl.BlockSpec((bm, No), lambda i: (i, 0)),
        out_shape=jax.ShapeDtypeStruct((M, No), jnp.float32),
    )(x, w, b2)


def _add2_body(x_ref, o_ref):
    o_ref[...] = x_ref[0] + x_ref[1]


def _add2(x, bm=1000):
    _, N, D = x.shape
    return pl.pallas_call(
        _add2_body,
        grid=(N // bm,),
        in_specs=[pl.BlockSpec((2, bm, D), lambda i: (0, i, 0))],
        out_specs=pl.BlockSpec((bm, D), lambda i: (i, 0)),
        out_shape=jax.ShapeDtypeStruct((N, D), jnp.float32),
    )(x)


def _pass_a(E, oe, with_ef):
    Hoe = H * oe
    nv = Hoe // 16
    noe = oe // 16
    nb = E // EB
    mesh = plsc.VectorSubcoreMesh(core_axis_name="c", subcore_axis_name="s")

    out_type = [jax.ShapeDtypeStruct((E, 16), jnp.float32)]
    if with_ef:
        out_type.append(jax.ShapeDtypeStruct((E, oe), jnp.float32))

    def body(pni_h, pnj_h, g_h, src_h, dst_h, gidx_h, attn_h,
             *outs_and_scratch):
        if with_ef:
            e_h, ef_h = outs_and_scratch[0], outs_and_scratch[1]
            rest = outs_and_scratch[2:]
        else:
            e_h = outs_and_scratch[0]
            ef_h = None
            rest = outs_and_scratch[1:]
        (src_v, dst_v, gidx_v, ni_v, nj_v, g_v, e_v, ef_v, attn_v, tr_v, sem) = rest
        wid = lax.axis_index("s") * NC + lax.axis_index("c")
        pltpu.sync_copy(attn_h, attn_v)
        for r in range(H, 16):
            tr_v[r, :] = jnp.zeros((16,), jnp.float32)
        nblk_w = (nb - wid + NW - 1) // NW

        def blk(k, _):
            b = wid + k * NW
            base = b * EB
            pltpu.sync_copy(src_h.at[pl.ds(base, EB)], src_v)
            pltpu.sync_copy(dst_h.at[pl.ds(base, EB)], dst_v)
            pltpu.sync_copy(gidx_h.at[pl.ds(base, EB)], gidx_v)
            c1 = pltpu.async_copy(pni_h.at[src_v], ni_v, sem)
            c2 = pltpu.async_copy(pnj_h.at[dst_v], nj_v, sem)
            c3 = pltpu.async_copy(g_h.at[gidx_v], g_v, sem)
            c1.wait()
            c2.wait()
            c3.wait()

            def edge(i, _):
                paccs = [jnp.zeros((16,), jnp.float32) for _ in range(H)]
                efaccs = [jnp.zeros((16,), jnp.float32) for _ in range(noe)]
                for v in range(nv):
                    sl = pl.ds(v * 16, 16)
                    x = ni_v[i, sl] + nj_v[i, sl] + g_v[i, sl]
                    f = jnp.maximum(x, 0.01 * x)
                    paccs[v // noe] = paccs[v // noe] + f * attn_v[sl]
                    if with_ef:
                        efaccs[v % noe] = efaccs[v % noe] + f
                for hh in range(H):
                    tr_v[hh, :] = paccs[hh]
                lane = lax.iota(jnp.int32, 16)
                erow = jnp.zeros((16,), jnp.float32)
                for j in range(16):
                    erow = erow + plsc.load_gather(
                        tr_v, [lane, jnp.full((16,), j, jnp.int32)])
                e_v[i, :] = erow
                if with_ef:
                    for j in range(noe):
                        ef_v[i, pl.ds(j * 16, 16)] = efaccs[j]
                return 0

            lax.fori_loop(0, EB, edge, 0)
            pltpu.sync_copy(e_v, e_h.at[pl.ds(base, EB)])
            if with_ef:
                pltpu.sync_copy(ef_v, ef_h.at[pl.ds(base, EB)])
            return 0

        lax.fori_loop(0, nblk_w, blk, 0)

    return pl.kernel(
        body,
        out_type=tuple(out_type) if with_ef else out_type[0],
        mesh=mesh,
        compiler_params=pltpu.CompilerParams(needs_layout_passes=False, use_tc_tiling_on_sc=False),
        scratch_types=[
            pltpu.VMEM((EB,), jnp.int32),
            pltpu.VMEM((EB,), jnp.int32),
            pltpu.VMEM((EB,), jnp.int32),
            pltpu.VMEM((EB, Hoe), jnp.float32),
            pltpu.VMEM((EB, Hoe), jnp.float32),
            pltpu.VMEM((EB, Hoe), jnp.float32),
            pltpu.VMEM((EB, 16), jnp.float32),
            pltpu.VMEM((EB, oe), jnp.float32),
            pltpu.VMEM((Hoe,), jnp.float32),
            pltpu.VMEM((16, 16), jnp.float32),
            pltpu.SemaphoreType.DMA,
        ],
    )


def _pass_b(E, N):
    nb = E // EB
    rpt = N // NS
    nz = rpt // ZR
    mesh = plsc.VectorSubcoreMesh(core_axis_name="c", subcore_axis_name="s")

    def body(e_h, dst_h, ee_h, den_h,
             dst_v, e_v, m_v, ee_v, z_v, den_sh, sem):
        cid = lax.axis_index("c")
        sid = lax.axis_index("s")
        wid = sid * NC + cid

        def zz(j, _):
            z_v[j, :] = jnp.zeros((16,), jnp.float32)
            return 0

        lax.fori_loop(0, ZR, zz, 0)
        for t in range(nz):
            pltpu.sync_copy(z_v, den_sh.at[pl.ds(sid * rpt + t * ZR, ZR)])
        plsc.subcore_barrier()

        nblk_w = (nb - wid + NW - 1) // NW

        def blk(k, _):
            b = wid + k * NW
            base = b * EB
            pltpu.sync_copy(dst_h.at[pl.ds(base, EB)], dst_v)
            pltpu.sync_copy(e_h.at[pl.ds(base, EB)], e_v)
            pltpu.async_copy(e_h.at[dst_v], m_v, sem).wait()

            def edge(i, _):
                ee_v[i, :] = jnp.exp(e_v[i, :] - m_v[i, :])
                return 0

            lax.fori_loop(0, EB, edge, 0)
            pltpu.sync_copy(ee_v, ee_h.at[pl.ds(base, EB)])
            pltpu.sync_copy(ee_v, den_sh.at[dst_v], add=True)
            return 0

        lax.fori_loop(0, nblk_w, blk, 0)
        plsc.subcore_barrier()
        for t in range(nz):
            off = sid * rpt + t * ZR
            pltpu.sync_copy(den_sh.at[pl.ds(off, ZR)], z_v)
            pltpu.sync_copy(z_v, den_h.at[cid, pl.ds(off, ZR)])

    return pl.kernel(
        body,
        out_type=(jax.ShapeDtypeStruct((E, 16), jnp.float32),
                  jax.ShapeDtypeStruct((NC, N, 16), jnp.float32)),
        mesh=mesh,
        compiler_params=pltpu.CompilerParams(needs_layout_passes=False, use_tc_tiling_on_sc=False),
        scratch_types=[
            pltpu.VMEM((EB,), jnp.int32),
            pltpu.VMEM((EB, 16), jnp.float32),
            pltpu.VMEM((EB, 16), jnp.float32),
            pltpu.VMEM((EB, 16), jnp.float32),
            pltpu.VMEM((ZR, 16), jnp.float32),
            pltpu.VMEM_SHARED((N, 16), jnp.float32),
            pltpu.SemaphoreType.DMA,
        ],
    )


def _pass_c(E, N, on):
    Hon = H * on
    non = on // 16
    nb = E // EB
    rpt = N // NS
    nz = rpt // ZR
    mesh = plsc.VectorSubcoreMesh(core_axis_name="c", subcore_axis_name="s")

    def body(ee_h, src_h, dst_h, den_h, ph_h, acc_h,
             src_v, dst_v, ee_v, den_v, a_v, ph_v, c_v, z_v, acc_sh, sem):
        cid = lax.axis_index("c")
        sid = lax.axis_index("s")
        wid = sid * NC + cid

        def zz(j, _):
            for j2 in range(non):
                z_v[j, pl.ds(j2 * 16, 16)] = jnp.zeros((16,), jnp.float32)
            return 0

        lax.fori_loop(0, ZR, zz, 0)
        for t in range(nz):
            pltpu.sync_copy(z_v, acc_sh.at[pl.ds(sid * rpt + t * ZR, ZR)])
        plsc.subcore_barrier()

        nblk_w = (nb - wid + NW - 1) // NW

        def blk(k, _):
            b = wid + k * NW
            base = b * EB
            pltpu.sync_copy(src_h.at[pl.ds(base, EB)], src_v)
            pltpu.sync_copy(dst_h.at[pl.ds(base, EB)], dst_v)
            pltpu.sync_copy(ee_h.at[pl.ds(base, EB)], ee_v)
            c1 = pltpu.async_copy(den_h.at[dst_v], den_v, sem)
            c2 = pltpu.async_copy(ph_h.at[src_v], ph_v, sem)
            c1.wait()
            c2.wait()

            def edge(i, _):
                arow = ee_v[i, :] / (den_v[i, :] + 1e-16)
                caccs = [jnp.zeros((16,), jnp.float32) for _ in range(non)]
                for hh in range(H):
                    a_s = arow[hh]
                    for j in range(non):
                        caccs[j] = caccs[j] + a_s * ph_v[i, pl.ds(hh * on + j * 16, 16)]
                for j in range(non):
                    c_v[i, pl.ds(j * 16, 16)] = caccs[j]
                return 0

            lax.fori_loop(0, EB, edge, 0)
            pltpu.sync_copy(c_v, acc_sh.at[dst_v], add=True)
            return 0

        lax.fori_loop(0, nblk_w, blk, 0)
        plsc.subcore_barrier()
        for t in range(nz):
            off = sid * rpt + t * ZR
            pltpu.sync_copy(acc_sh.at[pl.ds(off, ZR)], z_v)
            pltpu.sync_copy(z_v, acc_h.at[cid, pl.ds(off, ZR)])

    return pl.kernel(
        body,
        out_type=jax.ShapeDtypeStruct((NC, N, on), jnp.float32),
        mesh=mesh,
        compiler_params=pltpu.CompilerParams(needs_layout_passes=False, use_tc_tiling_on_sc=False),
        scratch_types=[
            pltpu.VMEM((EB,), jnp.int32),
            pltpu.VMEM((EB,), jnp.int32),
            pltpu.VMEM((EB, 16), jnp.float32),
            pltpu.VMEM((EB, 16), jnp.float32),
            pltpu.VMEM((1, 16), jnp.float32),
            pltpu.VMEM((EB, Hon), jnp.float32),
            pltpu.VMEM((EB, on), jnp.float32),
            pltpu.VMEM((ZR, on), jnp.float32),
            pltpu.VMEM_SHARED((N, on), jnp.float32),
            pltpu.SemaphoreType.DMA,
        ],
    )


def _layer(nfeats, grows, gidx, src, dst, N, E, Wnode, bnode, Wni, Wfij, Wnj,
           attn, bias, oe, on, with_ef):
    pni = _matmul(nfeats, Wni)
    pnj = _matmul(nfeats, Wnj)
    ph = _matmul(nfeats, Wnode, bnode)
    g = _matmul(grows, Wfij, bias)
    attn_flat = jnp.reshape(attn, (H * oe,))
    if with_ef:
        e_arr, ef_arr = _pass_a(E, oe, True)(pni, pnj, g, src, dst, gidx, attn_flat)
    else:
        e_arr = _pass_a(E, oe, False)(pni, pnj, g, src, dst, gidx, attn_flat)
        ef_arr = None
    ee_arr, den2 = _pass_b(E, N)(e_arr, dst)
    den = _add2(den2)
    acc2 = _pass_c(E, N, on)(ee_arr, src, dst, den, ph)
    out = _add2(acc2)
    return out, ef_arr


def kernel(node_feats, edge_feats, node_path, edge_path, edge_index_lg, edge_index_gg,
           Wnode1, bnode1, Wni1, Wfij1, Wnj1, attn1, b1,
           Wnode2a, bnode2a, Wni2a, Wfij2a, Wnj2a, attn2a, b2a,
           Wnode2b, bnode2b, Wni2b, Wfij2b, Wnj2b, attn2b, b2b):
    N_LG, E_LG = node_feats.shape[0], edge_index_lg.shape[1]
    N_GG, E_GG = node_path.shape[0], edge_index_gg.shape[1]
    src_g, dst_g = edge_index_gg[0], edge_index_gg[1]
    src_l, dst_l = edge_index_lg[0], edge_index_lg[1]
    iota_gg = jnp.arange(E_GG, dtype=jnp.int32)
    iota_lg = jnp.arange(E_LG, dtype=jnp.int32)
    half_lg = iota_lg // 2

    y2, _ = _layer(node_path, edge_path, iota_gg, src_g, dst_g, N_GG, E_GG,
                   Wnode1, bnode1, Wni1, Wfij1, Wnj1, attn1, b1, INF, BOND,
                   with_ef=False)
    nf, ef = _layer(node_feats, y2, half_lg, src_l, dst_l, N_LG, E_LG,
                    Wnode2a, bnode2a, Wni2a, Wfij2a, Wnj2a, attn2a, b2a,
                    BOND, ATOM, with_ef=True)
    nf, ef = _layer(nf, ef, iota_lg, src_l, dst_l, N_LG, E_LG,
                    Wnode2b, bnode2b, Wni2b, Wfij2b, Wnj2b, attn2b, b2b,
                    BOND, ATOM, with_ef=True)
    return nf, ef


# speedup vs baseline: 11.5118x; 1.4232x over previous
import jax
import jax.numpy as jnp
from jax import lax
from jax.experimental import pallas as pl
from jax.experimental.pallas import tpu as pltpu
from jax.experimental.pallas import tpu_sc as plsc

H = 8
INF = 16
BOND = 64
ATOM = 128
NC = 2
NS = 16
NW = NC * NS
ZR = 25

_SC_PARAMS = pltpu.CompilerParams(
    needs_layout_passes=False, use_tc_tiling_on_sc=False)


def _mm_body(x_ref, w_ref, b_ref, o_ref):
    o_ref[...] = lax.dot_general(
        x_ref[...], w_ref[...], (((1,), (1,)), ((), ())),
        preferred_element_type=jnp.float32) + b_ref[...]


def _matmul(x, w, b=None, bm=1000):
    M, K = x.shape
    No = w.shape[0]
    b2 = jnp.zeros((1, No), jnp.float32) if b is None else jnp.reshape(b, (1, No))
    return pl.pallas_call(
        _mm_body,
        grid=(M // bm,),
        in_specs=[pl.BlockSpec((bm, K), lambda i: (i, 0)),
                  pl.BlockSpec((No, K), lambda i: (0, 0)),
                  pl.BlockSpec((1, No), lambda i: (0, 0))],
        out_specs=pl.BlockSpec((bm, No), lambda i: (i, 0)),
        out_shape=jax.ShapeDtypeStruct((M, No), jnp.float32),
    )(x, w, b2)


def _add2_body(x_ref, o_ref):
    o_ref[...] = x_ref[0] + x_ref[1]


def _add2(x, bm=1000):
    _, N, D = x.shape
    return pl.pallas_call(
        _add2_body,
        grid=(N // bm,),
        in_specs=[pl.BlockSpec((2, bm, D), lambda i: (0, i, 0))],
        out_specs=pl.BlockSpec((bm, D), lambda i: (i, 0)),
        out_shape=jax.ShapeDtypeStruct((N, D), jnp.float32),
    )(x)


def _catdm_body(d_ref, m_ref, o_ref):
    o_ref[:, 0:16] = d_ref[0] + d_ref[1]
    o_ref[:, 16:32] = m_ref[...]


def _cat_den_m(den2, m, bm=1000):
    _, N, _ = den2.shape
    return pl.pallas_call(
        _catdm_body,
        grid=(N // bm,),
        in_specs=[pl.BlockSpec((2, bm, 16), lambda i: (0, i, 0)),
                  pl.BlockSpec((bm, 16), lambda i: (i, 0))],
        out_specs=pl.BlockSpec((bm, 32), lambda i: (i, 0)),
        out_shape=jax.ShapeDtypeStruct((N, 32), jnp.float32),
    )(den2, m)


_MESH = plsc.VectorSubcoreMesh(core_axis_name="c", subcore_axis_name="s")


def _zero_rows(z_v, nvec):
    def zz(j, _):
        for j2 in range(nvec):
            z_v[j, pl.ds(j2 * 16, 16)] = jnp.zeros((16,), jnp.float32)
        return 0
    lax.fori_loop(0, ZR, zz, 0)


def _pass_a(E, oe, with_ef, linear_g, EB):
    Hoe = H * oe
    nv = Hoe // 16
    noe = oe // 16
    nb = E // EB
    kmax = (nb + NW - 1) // NW
    kp = (kmax + 1) // 2

    out_type = [jax.ShapeDtypeStruct((E, 16), jnp.float32)]
    if with_ef:
        out_type.append(jax.ShapeDtypeStruct((E, oe), jnp.float32))

    def body(pni_h, pnj_h, g_h, src_h, dst_h, gidx_h, attn_h,
             *outs_and_scratch):
        if with_ef:
            e_h, ef_h = outs_and_scratch[0], outs_and_scratch[1]
            rest = outs_and_scratch[2:]
        else:
            e_h = outs_and_scratch[0]
            ef_h = None
            rest = outs_and_scratch[1:]
        (src_v, dst_v, gidx_v, ni_v, nj_v, g_v, e_v, ef_v, attn_v, tr_v,
         sem0, sem1) = rest
        sems = (sem0, sem1)
        wid = lax.axis_index("s") * NC + lax.axis_index("c")
        pltpu.sync_copy(attn_h, attn_v)
        for r in range(H, 16):
            tr_v[r, :] = jnp.zeros((16,), jnp.float32)
        kw = (nb - wid + NW - 1) // NW

        def issue(slot, k):
            @pl.when(k < kw)
            def _():
                base = (wid + k * NW) * EB
                pltpu.sync_copy(src_h.at[pl.ds(base, EB)], src_v.at[slot])
                pltpu.sync_copy(dst_h.at[pl.ds(base, EB)], dst_v.at[slot])
                pltpu.async_copy(pni_h.at[src_v.at[slot]], ni_v.at[slot], sems[slot])
                pltpu.async_copy(pnj_h.at[dst_v.at[slot]], nj_v.at[slot], sems[slot])
                if linear_g:
                    pltpu.async_copy(g_h.at[pl.ds(base, EB)], g_v.at[slot], sems[slot])
                else:
                    pltpu.sync_copy(gidx_h.at[pl.ds(base, EB)], gidx_v.at[slot])
                    pltpu.async_copy(g_h.at[gidx_v.at[slot]], g_v.at[slot], sems[slot])

        def compute(slot, k):
            @pl.when(k < kw)
            def _():
                base = (wid + k * NW) * EB
                pltpu.make_async_copy(pni_h.at[src_v.at[slot]], ni_v.at[slot], sems[slot]).wait()
                pltpu.make_async_copy(pni_h.at[src_v.at[slot]], nj_v.at[slot], sems[slot]).wait()
                pltpu.make_async_copy(pni_h.at[src_v.at[slot]], g_v.at[slot], sems[slot]).wait()

                def edge(i, _):
                    paccs = [jnp.zeros((16,), jnp.float32) for _ in range(H)]
                    efaccs = [jnp.zeros((16,), jnp.float32) for _ in range(noe)]
                    for v in range(nv):
                        sl = pl.ds(v * 16, 16)
                        x = ni_v[slot, i, sl] + nj_v[slot, i, sl] + g_v[slot, i, sl]
                        f = jnp.maximum(x, 0.01 * x)
                        paccs[v // noe] = paccs[v // noe] + f * attn_v[sl]
                        if with_ef:
                            efaccs[v % noe] = efaccs[v % noe] + f
                    for hh in range(H):
                        tr_v[hh, :] = paccs[hh]
                    lane = lax.iota(jnp.int32, 16)
                    cols = [plsc.load_gather(
                        tr_v, [lane, jnp.full((16,), j, jnp.int32)])
                        for j in range(16)]
                    while len(cols) > 1:
                        cols = [cols[2 * t] + cols[2 * t + 1]
                                for t in range(len(cols) // 2)]
                    e_v[i, :] = cols[0]
                    if with_ef:
                        for j in range(noe):
                            ef_v[i, pl.ds(j * 16, 16)] = efaccs[j]
                    return 0

                lax.fori_loop(0, EB, edge, 0)
                pltpu.sync_copy(e_v, e_h.at[pl.ds(base, EB)])
                if with_ef:
                    pltpu.sync_copy(ef_v, ef_h.at[pl.ds(base, EB)])

        issue(0, 0)

        def pairbody(k2, _):
            k0 = 2 * k2
            k1 = 2 * k2 + 1
            issue(1, k1)
            compute(0, k0)
            issue(0, k1 + 1)
            compute(1, k1)
            return 0

        lax.fori_loop(0, kp, pairbody, 0)

    return pl.kernel(
        body,
        out_type=tuple(out_type) if with_ef else out_type[0],
        mesh=_MESH,
        compiler_params=_SC_PARAMS,
        scratch_types=[
            pltpu.VMEM((2, EB), jnp.int32),
            pltpu.VMEM((2, EB), jnp.int32),
            pltpu.VMEM((2, EB), jnp.int32),
            pltpu.VMEM((2, EB, Hoe), jnp.float32),
            pltpu.VMEM((2, EB, Hoe), jnp.float32),
            pltpu.VMEM((2, EB, Hoe), jnp.float32),
            pltpu.VMEM((EB, 16), jnp.float32),
            pltpu.VMEM((EB, oe), jnp.float32),
            pltpu.VMEM((Hoe,), jnp.float32),
            pltpu.VMEM((16, 16), jnp.float32),
            pltpu.SemaphoreType.DMA,
            pltpu.SemaphoreType.DMA,
        ],
    )


def _pass_b(E, N, EB):
    nb = E // EB
    rpt = N // NS
    nz = rpt // ZR
    kmax = (nb + NW - 1) // NW
    kp = (kmax + 1) // 2

    def body(e_h, dst_h, den_h,
             dst_v, e_v, m_v, ee_v, z_v, den_sh, sem0, sem1):
        sems = (sem0, sem1)
        cid = lax.axis_index("c")
        sid = lax.axis_index("s")
        wid = sid * NC + cid
        _zero_rows(z_v, 1)
        for t in range(nz):
            pltpu.sync_copy(z_v, den_sh.at[pl.ds(sid * rpt + t * ZR, ZR)])
        plsc.subcore_barrier()
        kw = (nb - wid + NW - 1) // NW

        def issue(slot, k):
            @pl.when(k < kw)
            def _():
                base = (wid + k * NW) * EB
                pltpu.sync_copy(dst_h.at[pl.ds(base, EB)], dst_v.at[slot])
                pltpu.async_copy(e_h.at[pl.ds(base, EB)], e_v.at[slot], sems[slot])
                pltpu.async_copy(e_h.at[dst_v.at[slot]], m_v.at[slot], sems[slot])

        def compute(slot, k):
            @pl.when(k < kw)
            def _():
                pltpu.make_async_copy(e_h.at[dst_v.at[slot]], e_v.at[slot], sems[slot]).wait()
                pltpu.make_async_copy(e_h.at[dst_v.at[slot]], m_v.at[slot], sems[slot]).wait()

                def edge(i, _):
                    ee_v[i, :] = jnp.exp(e_v[slot, i, :] - m_v[slot, i, :])
                    return 0

                lax.fori_loop(0, EB, edge, 0)
                pltpu.sync_copy(ee_v, den_sh.at[dst_v.at[slot]], add=True)

        issue(0, 0)

        def pairbody(k2, _):
            k1 = 2 * k2 + 1
            issue(1, k1)
            compute(0, 2 * k2)
            issue(0, k1 + 1)
            compute(1, k1)
            return 0

        lax.fori_loop(0, kp, pairbody, 0)
        plsc.subcore_barrier()
        for t in range(nz):
            off = sid * rpt + t * ZR
            pltpu.sync_copy(den_sh.at[pl.ds(off, ZR)], z_v)
            pltpu.sync_copy(z_v, den_h.at[cid, pl.ds(off, ZR)])

    return pl.kernel(
        body,
        out_type=jax.ShapeDtypeStruct((NC, N, 16), jnp.float32),
        mesh=_MESH,
        compiler_params=_SC_PARAMS,
        scratch_types=[
            pltpu.VMEM((2, EB), jnp.int32),
            pltpu.VMEM((2, EB, 16), jnp.float32),
            pltpu.VMEM((2, EB, 16), jnp.float32),
            pltpu.VMEM((EB, 16), jnp.float32),
            pltpu.VMEM((ZR, 16), jnp.float32),
            pltpu.VMEM_SHARED((N, 16), jnp.float32),
            pltpu.SemaphoreType.DMA,
            pltpu.SemaphoreType.DMA,
        ],
    )


def _pass_c(E, N, on, EB):
    Hon = H * on
    non = on // 16
    nb = E // EB
    rpt = N // NS
    nz = rpt // ZR
    kmax = (nb + NW - 1) // NW
    kp = (kmax + 1) // 2

    def body(e_h, src_h, dst_h, dm_h, ph_h, acc_h,
             src_v, dst_v, e_v, dm_v, ph_v, c_v, z_v, acc_sh, sem0, sem1):
        sems = (sem0, sem1)
        cid = lax.axis_index("c")
        sid = lax.axis_index("s")
        wid = sid * NC + cid
        _zero_rows(z_v, non)
        for t in range(nz):
            pltpu.sync_copy(z_v, acc_sh.at[pl.ds(sid * rpt + t * ZR, ZR)])
        plsc.subcore_barrier()
        kw = (nb - wid + NW - 1) // NW

        def issue(slot, k):
            @pl.when(k < kw)
            def _():
                base = (wid + k * NW) * EB
                pltpu.sync_copy(src_h.at[pl.ds(base, EB)], src_v.at[slot])
                pltpu.sync_copy(dst_h.at[pl.ds(base, EB)], dst_v.at[slot])
                pltpu.async_copy(e_h.at[pl.ds(base, EB)], e_v.at[slot], sems[slot])
                pltpu.async_copy(dm_h.at[dst_v.at[slot]], dm_v.at[slot], sems[slot])
                pltpu.async_copy(ph_h.at[src_v.at[slot]], ph_v.at[slot], sems[slot])

        def compute(slot, k):
            @pl.when(k < kw)
            def _():
                pltpu.make_async_copy(e_h.at[dst_v.at[slot]], e_v.at[slot], sems[slot]).wait()
                pltpu.make_async_copy(dm_h.at[dst_v.at[slot]], dm_v.at[slot], sems[slot]).wait()
                pltpu.make_async_copy(ph_h.at[src_v.at[slot]], ph_v.at[slot], sems[slot]).wait()

                def edge(i, _):
                    ee = jnp.exp(e_v[slot, i, :] - dm_v[slot, i, pl.ds(16, 16)])
                    arow = ee / (dm_v[slot, i, pl.ds(0, 16)] + 1e-16)
                    caccs = [jnp.zeros((16,), jnp.float32) for _ in range(non)]
                    for hh in range(H):
                        a_s = arow[hh]
                        for j in range(non):
                            caccs[j] = caccs[j] + a_s * ph_v[slot, i, pl.ds(hh * on + j * 16, 16)]
                    for j in range(non):
                        c_v[i, pl.ds(j * 16, 16)] = caccs[j]
                    return 0

                lax.fori_loop(0, EB, edge, 0)
                pltpu.sync_copy(c_v, acc_sh.at[dst_v.at[slot]], add=True)

        issue(0, 0)

        def pairbody(k2, _):
            k1 = 2 * k2 + 1
            issue(1, k1)
            compute(0, 2 * k2)
            issue(0, k1 + 1)
            compute(1, k1)
            return 0

        lax.fori_loop(0, kp, pairbody, 0)
        plsc.subcore_barrier()
        for t in range(nz):
            off = sid * rpt + t * ZR
            pltpu.sync_copy(acc_sh.at[pl.ds(off, ZR)], z_v)
            pltpu.sync_copy(z_v, acc_h.at[cid, pl.ds(off, ZR)])

    return pl.kernel(
        body,
        out_type=jax.ShapeDtypeStruct((NC, N, on), jnp.float32),
        mesh=_MESH,
        compiler_params=_SC_PARAMS,
        scratch_types=[
            pltpu.VMEM((2, EB), jnp.int32),
            pltpu.VMEM((2, EB), jnp.int32),
            pltpu.VMEM((2, EB, 16), jnp.float32),
            pltpu.VMEM((2, EB, 32), jnp.float32),
            pltpu.VMEM((2, EB, Hon), jnp.float32),
            pltpu.VMEM((EB, on), jnp.float32),
            pltpu.VMEM((ZR, on), jnp.float32),
            pltpu.VMEM_SHARED((N, on), jnp.float32),
            pltpu.SemaphoreType.DMA,
            pltpu.SemaphoreType.DMA,
        ],
    )


def _layer(nfeats, grows, gidx, src, dst, N, E, Wnode, bnode, Wni, Wfij, Wnj,
           attn, bias, oe, on, with_ef, linear_g, eba, ebc):
    pni = _matmul(nfeats, Wni)
    pnj = _matmul(nfeats, Wnj)
    ph = _matmul(nfeats, Wnode, bnode)
    g = _matmul(grows, Wfij, bias)
    attn_flat = jnp.reshape(attn, (H * oe,))
    gidx_in = src if gidx is None else gidx
    if with_ef:
        e_arr, ef_arr = _pass_a(E, oe, True, linear_g, eba)(
            pni, pnj, g, src, dst, gidx_in, attn_flat)
    else:
        e_arr = _pass_a(E, oe, False, linear_g, eba)(
            pni, pnj, g, src, dst, gidx_in, attn_flat)
        ef_arr = None
    den2 = _pass_b(E, N, 32)(e_arr, dst)
    dm = _cat_den_m(den2, e_arr[:N])
    acc2 = _pass_c(E, N, on, ebc)(e_arr, src, dst, dm, ph)
    out = _add2(acc2)
    return out, ef_arr


def kernel(node_feats, edge_feats, node_path, edge_path, edge_index_lg, edge_index_gg,
           Wnode1, bnode1, Wni1, Wfij1, Wnj1, attn1, b1,
           Wnode2a, bnode2a, Wni2a, Wfij2a, Wnj2a, attn2a, b2a,
           Wnode2b, bnode2b, Wni2b, Wfij2b, Wnj2b, attn2b, b2b):
    N_LG, E_LG = node_feats.shape[0], edge_index_lg.shape[1]
    N_GG, E_GG = node_path.shape[0], edge_index_gg.shape[1]
    src_g, dst_g = edge_index_gg[0], edge_index_gg[1]
    src_l, dst_l = edge_index_lg[0], edge_index_lg[1]
    half_lg = jnp.arange(E_LG, dtype=jnp.int32) // 2

    y2, _ = _layer(node_path, edge_path, None, src_g, dst_g, N_GG, E_GG,
                   Wnode1, bnode1, Wni1, Wfij1, Wnj1, attn1, b1, INF, BOND,
                   with_ef=False, linear_g=True, eba=64, ebc=32)
    nf, ef = _layer(node_feats, y2, half_lg, src_l, dst_l, N_LG, E_LG,
                    Wnode2a, bnode2a, Wni2a, Wfij2a, Wnj2a, attn2a, b2a,
                    BOND, ATOM, with_ef=True, linear_g=False, eba=32, ebc=16)
    nf, ef = _layer(nf, ef, None, src_l, dst_l, N_LG, E_LG,
                    Wnode2b, bnode2b, Wni2b, Wfij2b, Wnj2b, attn2b, b2b,
                    BOND, ATOM, with_ef=True, linear_g=True, eba=32, ebc=16)
    return nf, ef


# speedup vs baseline: 11.9787x; 1.0406x over previous
import jax
import jax.numpy as jnp
from jax import lax
from jax.experimental import pallas as pl
from jax.experimental.pallas import tpu as pltpu
from jax.experimental.pallas import tpu_sc as plsc

H = 8
INF = 16
BOND = 64
ATOM = 128
NC = 2
NS = 16
NW = NC * NS
ZR = 25

_SC_PARAMS = pltpu.CompilerParams(
    needs_layout_passes=False, use_tc_tiling_on_sc=False)


def _mm_body(x_ref, w_ref, b_ref, o_ref):
    o_ref[...] = lax.dot_general(
        x_ref[...], w_ref[...], (((1,), (1,)), ((), ())),
        preferred_element_type=jnp.float32) + b_ref[...]


def _matmul(x, w, b=None, bm=1000):
    M, K = x.shape
    No = w.shape[0]
    b2 = jnp.zeros((1, No), jnp.float32) if b is None else jnp.reshape(b, (1, No))
    return pl.pallas_call(
        _mm_body,
        grid=(M // bm,),
        in_specs=[pl.BlockSpec((bm, K), lambda i: (i, 0)),
                  pl.BlockSpec((No, K), lambda i: (0, 0)),
                  pl.BlockSpec((1, No), lambda i: (0, 0))],
        out_specs=pl.BlockSpec((bm, No), lambda i: (i, 0)),
        out_shape=jax.ShapeDtypeStruct((M, No), jnp.float32),
    )(x, w, b2)


def _add2_body(x_ref, o_ref):
    o_ref[...] = x_ref[0] + x_ref[1]


def _add2(x, bm=1000):
    _, N, D = x.shape
    return pl.pallas_call(
        _add2_body,
        grid=(N // bm,),
        in_specs=[pl.BlockSpec((2, bm, D), lambda i: (0, i, 0))],
        out_specs=pl.BlockSpec((bm, D), lambda i: (i, 0)),
        out_shape=jax.ShapeDtypeStruct((N, D), jnp.float32),
    )(x)


def _catdm_body(d_ref, m_ref, o_ref):
    o_ref[:, 0:16] = d_ref[0] + d_ref[1]
    o_ref[:, 16:32] = m_ref[...]


def _cat_den_m(den2, m, bm=1000):
    _, N, _ = den2.shape
    return pl.pallas_call(
        _catdm_body,
        grid=(N // bm,),
        in_specs=[pl.BlockSpec((2, bm, 16), lambda i: (0, i, 0)),
                  pl.BlockSpec((bm, 16), lambda i: (i, 0))],
        out_specs=pl.BlockSpec((bm, 32), lambda i: (i, 0)),
        out_shape=jax.ShapeDtypeStruct((N, 32), jnp.float32),
    )(den2, m)


_MESH = plsc.VectorSubcoreMesh(core_axis_name="c", subcore_axis_name="s")


def _zero_rows(z_v, nvec, zr):
    def zz(j, _):
        for j2 in range(nvec):
            z_v[j, pl.ds(j2 * 16, 16)] = jnp.zeros((16,), jnp.float32)
        return 0
    lax.fori_loop(0, zr, zz, 0)


def _pass_a(E, oe, with_ef, linear_g, EB):
    Hoe = H * oe
    nv = Hoe // 16
    noe = oe // 16
    nb = E // EB
    kmax = (nb + NW - 1) // NW
    kp = (kmax + 1) // 2

    out_type = [jax.ShapeDtypeStruct((E, 16), jnp.float32)]
    if with_ef:
        out_type.append(jax.ShapeDtypeStruct((E, oe), jnp.float32))

    def body(pni_h, pnj_h, g_h, src_h, dst_h, gidx_h, attn_h,
             *outs_and_scratch):
        if with_ef:
            e_h, ef_h = outs_and_scratch[0], outs_and_scratch[1]
            rest = outs_and_scratch[2:]
        else:
            e_h = outs_and_scratch[0]
            ef_h = None
            rest = outs_and_scratch[1:]
        (src_v, dst_v, gidx_v, ni_v, nj_v, g_v, e_v, ef_v, attn_v, tr_v,
         sem0, sem1) = rest
        sems = (sem0, sem1)
        wid = lax.axis_index("s") * NC + lax.axis_index("c")
        pltpu.sync_copy(attn_h, attn_v)
        for r in range(H, 16):
            tr_v[r, :] = jnp.zeros((16,), jnp.float32)
        kw = (nb - wid + NW - 1) // NW

        def issue(slot, k):
            @pl.when(k < kw)
            def _():
                base = (wid + k * NW) * EB
                pltpu.sync_copy(src_h.at[pl.ds(base, EB)], src_v.at[slot])
                pltpu.sync_copy(dst_h.at[pl.ds(base, EB)], dst_v.at[slot])
                pltpu.async_copy(pni_h.at[src_v.at[slot]], ni_v.at[slot], sems[slot])
                pltpu.async_copy(pnj_h.at[dst_v.at[slot]], nj_v.at[slot], sems[slot])
                if linear_g:
                    pltpu.async_copy(g_h.at[pl.ds(base, EB)], g_v.at[slot], sems[slot])
                else:
                    pltpu.sync_copy(gidx_h.at[pl.ds(base, EB)], gidx_v.at[slot])
                    pltpu.async_copy(g_h.at[gidx_v.at[slot]], g_v.at[slot], sems[slot])

        def compute(slot, k):
            @pl.when(k < kw)
            def _():
                base = (wid + k * NW) * EB
                pltpu.make_async_copy(pni_h.at[src_v.at[slot]], ni_v.at[slot], sems[slot]).wait()
                pltpu.make_async_copy(pni_h.at[src_v.at[slot]], nj_v.at[slot], sems[slot]).wait()
                pltpu.make_async_copy(pni_h.at[src_v.at[slot]], g_v.at[slot], sems[slot]).wait()

                def edge(i, _):
                    paccs = [jnp.zeros((16,), jnp.float32) for _ in range(H)]
                    efaccs = [jnp.zeros((16,), jnp.float32) for _ in range(noe)]
                    for v in range(nv):
                        sl = pl.ds(v * 16, 16)
                        x = ni_v[slot, i, sl] + nj_v[slot, i, sl] + g_v[slot, i, sl]
                        f = jnp.maximum(x, 0.01 * x)
                        paccs[v // noe] = paccs[v // noe] + f * attn_v[sl]
                        if with_ef:
                            efaccs[v % noe] = efaccs[v % noe] + f
                    for hh in range(H):
                        tr_v[hh, :] = paccs[hh]
                    lane = lax.iota(jnp.int32, 16)
                    cols = [plsc.load_gather(
                        tr_v, [lane, jnp.full((16,), j, jnp.int32)])
                        for j in range(16)]
                    while len(cols) > 1:
                        cols = [cols[2 * t] + cols[2 * t + 1]
                                for t in range(len(cols) // 2)]
                    e_v[i, :] = cols[0]
                    if with_ef:
                        for j in range(noe):
                            ef_v[i, pl.ds(j * 16, 16)] = efaccs[j]
                    return 0

                lax.fori_loop(0, EB, edge, 0, unroll=2)
                pltpu.sync_copy(e_v, e_h.at[pl.ds(base, EB)])
                if with_ef:
                    pltpu.sync_copy(ef_v, ef_h.at[pl.ds(base, EB)])

        issue(0, 0)

        def pairbody(k2, _):
            k0 = 2 * k2
            k1 = 2 * k2 + 1
            issue(1, k1)
            compute(0, k0)
            issue(0, k1 + 1)
            compute(1, k1)
            return 0

        lax.fori_loop(0, kp, pairbody, 0)

    return pl.kernel(
        body,
        out_type=tuple(out_type) if with_ef else out_type[0],
        mesh=_MESH,
        compiler_params=_SC_PARAMS,
        scratch_types=[
            pltpu.VMEM((2, EB), jnp.int32),
            pltpu.VMEM((2, EB), jnp.int32),
            pltpu.VMEM((2, EB), jnp.int32),
            pltpu.VMEM((2, EB, Hoe), jnp.float32),
            pltpu.VMEM((2, EB, Hoe), jnp.float32),
            pltpu.VMEM((2, EB, Hoe), jnp.float32),
            pltpu.VMEM((EB, 16), jnp.float32),
            pltpu.VMEM((EB, oe), jnp.float32),
            pltpu.VMEM((Hoe,), jnp.float32),
            pltpu.VMEM((16, 16), jnp.float32),
            pltpu.SemaphoreType.DMA,
            pltpu.SemaphoreType.DMA,
        ],
    )


def _pass_b(E, N, EB, zr):
    nb = E // EB
    rpt = N // NS
    nz = rpt // zr
    kmax = (nb + NW - 1) // NW
    kp = (kmax + 1) // 2

    def body(e_h, dst_h, den_h,
             dst_v, e_v, m_v, ee_v, z_v, den_sh, sem0, sem1):
        sems = (sem0, sem1)
        cid = lax.axis_index("c")
        sid = lax.axis_index("s")
        wid = sid * NC + cid
        _zero_rows(z_v, 1, zr)
        for t in range(nz):
            pltpu.sync_copy(z_v, den_sh.at[pl.ds(sid * rpt + t * zr, zr)])
        plsc.subcore_barrier()
        kw = (nb - wid + NW - 1) // NW

        def issue(slot, k):
            @pl.when(k < kw)
            def _():
                base = (wid + k * NW) * EB
                pltpu.sync_copy(dst_h.at[pl.ds(base, EB)], dst_v.at[slot])
                pltpu.async_copy(e_h.at[pl.ds(base, EB)], e_v.at[slot], sems[slot])
                pltpu.async_copy(e_h.at[dst_v.at[slot]], m_v.at[slot], sems[slot])

        def compute(slot, k):
            @pl.when(k < kw)
            def _():
                pltpu.make_async_copy(e_h.at[dst_v.at[slot]], e_v.at[slot], sems[slot]).wait()
                pltpu.make_async_copy(e_h.at[dst_v.at[slot]], m_v.at[slot], sems[slot]).wait()

                def edge(i, _):
                    ee_v[i, :] = jnp.exp(e_v[slot, i, :] - m_v[slot, i, :])
                    return 0

                lax.fori_loop(0, EB, edge, 0)
                pltpu.sync_copy(ee_v, den_sh.at[dst_v.at[slot]], add=True)

        issue(0, 0)

        def pairbody(k2, _):
            k1 = 2 * k2 + 1
            issue(1, k1)
            compute(0, 2 * k2)
            issue(0, k1 + 1)
            compute(1, k1)
            return 0

        lax.fori_loop(0, kp, pairbody, 0)
        plsc.subcore_barrier()
        pltpu.sync_copy(den_sh.at[pl.ds(sid * rpt, rpt)],
                        den_h.at[cid, pl.ds(sid * rpt, rpt)])

    return pl.kernel(
        body,
        out_type=jax.ShapeDtypeStruct((NC, N, 16), jnp.float32),
        mesh=_MESH,
        compiler_params=_SC_PARAMS,
        scratch_types=[
            pltpu.VMEM((2, EB), jnp.int32),
            pltpu.VMEM((2, EB, 16), jnp.float32),
            pltpu.VMEM((2, EB, 16), jnp.float32),
            pltpu.VMEM((EB, 16), jnp.float32),
            pltpu.VMEM((zr, 16), jnp.float32),
            pltpu.VMEM_SHARED((N, 16), jnp.float32),
            pltpu.SemaphoreType.DMA,
            pltpu.SemaphoreType.DMA,
        ],
    )


def _pass_c(E, N, on, EB, zr):
    Hon = H * on
    non = on // 16
    nb = E // EB
    rpt = N // NS
    nz = rpt // zr
    kmax = (nb + NW - 1) // NW
    kp = (kmax + 1) // 2

    def body(e_h, src_h, dst_h, dm_h, ph_h, acc_h,
             src_v, dst_v, e_v, dm_v, ph_v, c_v, z_v, acc_sh, sem0, sem1):
        sems = (sem0, sem1)
        cid = lax.axis_index("c")
        sid = lax.axis_index("s")
        wid = sid * NC + cid
        _zero_rows(z_v, non, zr)
        for t in range(nz):
            pltpu.sync_copy(z_v, acc_sh.at[pl.ds(sid * rpt + t * zr, zr)])
        plsc.subcore_barrier()
        kw = (nb - wid + NW - 1) // NW

        def issue(slot, k):
            @pl.when(k < kw)
            def _():
                base = (wid + k * NW) * EB
                pltpu.sync_copy(src_h.at[pl.ds(base, EB)], src_v.at[slot])
                pltpu.sync_copy(dst_h.at[pl.ds(base, EB)], dst_v.at[slot])
                pltpu.async_copy(e_h.at[pl.ds(base, EB)], e_v.at[slot], sems[slot])
                pltpu.async_copy(dm_h.at[dst_v.at[slot]], dm_v.at[slot], sems[slot])
                pltpu.async_copy(ph_h.at[src_v.at[slot]], ph_v.at[slot], sems[slot])

        def compute(slot, k):
            @pl.when(k < kw)
            def _():
                pltpu.make_async_copy(e_h.at[dst_v.at[slot]], e_v.at[slot], sems[slot]).wait()
                pltpu.make_async_copy(dm_h.at[dst_v.at[slot]], dm_v.at[slot], sems[slot]).wait()
                pltpu.make_async_copy(ph_h.at[src_v.at[slot]], ph_v.at[slot], sems[slot]).wait()

                def edge(i, _):
                    ee = jnp.exp(e_v[slot, i, :] - dm_v[slot, i, pl.ds(16, 16)])
                    arow = ee / (dm_v[slot, i, pl.ds(0, 16)] + 1e-16)
                    caccs = [jnp.zeros((16,), jnp.float32) for _ in range(non)]
                    for hh in range(H):
                        a_s = arow[hh]
                        for j in range(non):
                            caccs[j] = caccs[j] + a_s * ph_v[slot, i, pl.ds(hh * on + j * 16, 16)]
                    for j in range(non):
                        c_v[i, pl.ds(j * 16, 16)] = caccs[j]
                    return 0

                lax.fori_loop(0, EB, edge, 0, unroll=2)
                pltpu.sync_copy(c_v, acc_sh.at[dst_v.at[slot]], add=True)

        issue(0, 0)

        def pairbody(k2, _):
            k1 = 2 * k2 + 1
            issue(1, k1)
            compute(0, 2 * k2)
            issue(0, k1 + 1)
            compute(1, k1)
            return 0

        lax.fori_loop(0, kp, pairbody, 0)
        plsc.subcore_barrier()
        pltpu.sync_copy(acc_sh.at[pl.ds(sid * rpt, rpt)],
                        acc_h.at[cid, pl.ds(sid * rpt, rpt)])

    return pl.kernel(
        body,
        out_type=jax.ShapeDtypeStruct((NC, N, on), jnp.float32),
        mesh=_MESH,
        compiler_params=_SC_PARAMS,
        scratch_types=[
            pltpu.VMEM((2, EB), jnp.int32),
            pltpu.VMEM((2, EB), jnp.int32),
            pltpu.VMEM((2, EB, 16), jnp.float32),
            pltpu.VMEM((2, EB, 32), jnp.float32),
            pltpu.VMEM((2, EB, Hon), jnp.float32),
            pltpu.VMEM((EB, on), jnp.float32),
            pltpu.VMEM((zr, on), jnp.float32),
            pltpu.VMEM_SHARED((N, on), jnp.float32),
            pltpu.SemaphoreType.DMA,
            pltpu.SemaphoreType.DMA,
        ],
    )


def _layer(nfeats, grows, gidx, src, dst, N, E, Wnode, bnode, Wni, Wfij, Wnj,
           attn, bias, oe, on, with_ef, linear_g, eba, ebc, zrc):
    pni = _matmul(nfeats, Wni)
    pnj = _matmul(nfeats, Wnj)
    ph = _matmul(nfeats, Wnode, bnode)
    g = _matmul(grows, Wfij, bias)
    attn_flat = jnp.reshape(attn, (H * oe,))
    gidx_in = src if gidx is None else gidx
    if with_ef:
        e_arr, ef_arr = _pass_a(E, oe, True, linear_g, eba)(
            pni, pnj, g, src, dst, gidx_in, attn_flat)
    else:
        e_arr = _pass_a(E, oe, False, linear_g, eba)(
            pni, pnj, g, src, dst, gidx_in, attn_flat)
        ef_arr = None
    den2 = _pass_b(E, N, 80, 625)(e_arr, dst)
    dm = _cat_den_m(den2, e_arr[:N])
    acc2 = _pass_c(E, N, on, ebc, zrc)(e_arr, src, dst, dm, ph)
    out = _add2(acc2)
    return out, ef_arr


def kernel(node_feats, edge_feats, node_path, edge_path, edge_index_lg, edge_index_gg,
           Wnode1, bnode1, Wni1, Wfij1, Wnj1, attn1, b1,
           Wnode2a, bnode2a, Wni2a, Wfij2a, Wnj2a, attn2a, b2a,
           Wnode2b, bnode2b, Wni2b, Wfij2b, Wnj2b, attn2b, b2b):
    N_LG, E_LG = node_feats.shape[0], edge_index_lg.shape[1]
    N_GG, E_GG = node_path.shape[0], edge_index_gg.shape[1]
    src_g, dst_g = edge_index_gg[0], edge_index_gg[1]
    src_l, dst_l = edge_index_lg[0], edge_index_lg[1]
    half_lg = jnp.arange(E_LG, dtype=jnp.int32) // 2

    y2, _ = _layer(node_path, edge_path, None, src_g, dst_g, N_GG, E_GG,
                   Wnode1, bnode1, Wni1, Wfij1, Wnj1, attn1, b1, INF, BOND,
                   with_ef=False, linear_g=True, eba=64, ebc=32, zrc=125)
    nf, ef = _layer(node_feats, y2, half_lg, src_l, dst_l, N_LG, E_LG,
                    Wnode2a, bnode2a, Wni2a, Wfij2a, Wnj2a, attn2a, b2a,
                    BOND, ATOM, with_ef=True, linear_g=False, eba=32, ebc=16, zrc=25)
    nf, ef = _layer(nf, ef, None, src_l, dst_l, N_LG, E_LG,
                    Wnode2b, bnode2b, Wni2b, Wfij2b, Wnj2b, attn2b, b2b,
                    BOND, ATOM, with_ef=True, linear_g=True, eba=32, ebc=16, zrc=25)
    return nf, ef
